# Initial kernel scaffold; baseline (speedup 1.0000x reference)
import jax
import jax.numpy as jnp
from jax.experimental import pallas as pl


def kernel(node_features, edge_index, edge_features, W1a, b1a, W2a, b2a, W1b, b1b, W2b, b2b):
    raise NotImplementedError("write your pallas kernel here")


# speedup vs baseline: 1.9412x; 1.9412x over previous
import functools

import jax
import jax.numpy as jnp
from jax import lax
from jax.experimental import pallas as pl
from jax.experimental.pallas import tpu as pltpu
from jax.experimental.pallas import tpu_sc as plsc

_E = 800000
_C_IN = 16
_C_HID = 64
_NW = 32
_CHUNK = 1024
_GSLICE = 128
_NSTEP = 25
_PER_W = _CHUNK * _NSTEP
_E_PAD = _PER_W * _NW


def _gather_body(nf_hbm, src_hbm, dst_hbm, srcout_hbm, dstout_hbm,
                 sidx, didx, srows, drows, sem_s, sem_d):
    wid = lax.axis_index("s") * 2 + lax.axis_index("c")
    row0 = wid * (_PER_W // _GSLICE)

    def step(c, carry):
        idx_row = row0 + c * (_CHUNK // _GSLICE)
        off = (wid * _PER_W + c * _CHUNK)
        pltpu.sync_copy(src_hbm.at[pl.ds(idx_row, _CHUNK // _GSLICE)], sidx)
        pltpu.sync_copy(dst_hbm.at[pl.ds(idx_row, _CHUNK // _GSLICE)], didx)
        copies = []
        for j in range(_CHUNK // _GSLICE):
            copies.append(pltpu.async_copy(
                nf_hbm.at[sidx.at[j]], srows.at[pl.ds(j * _GSLICE, _GSLICE)], sem_s))
            copies.append(pltpu.async_copy(
                nf_hbm.at[didx.at[j]], drows.at[pl.ds(j * _GSLICE, _GSLICE)], sem_d))
        for cp in copies:
            cp.wait()
        pltpu.sync_copy(srows, srcout_hbm.at[pl.ds(off, _CHUNK)])
        pltpu.sync_copy(drows, dstout_hbm.at[pl.ds(off, _CHUNK)])
        return carry

    lax.fori_loop(0, _NSTEP, step, 0)


def _sc_gather(node_features, src2d, dst2d):
    mesh = plsc.VectorSubcoreMesh(core_axis_name="c", subcore_axis_name="s")
    f = pl.kernel(
        _gather_body,
        out_type=(jax.ShapeDtypeStruct((_E_PAD, _C_IN), jnp.float32),
                  jax.ShapeDtypeStruct((_E_PAD, _C_IN), jnp.float32)),
        mesh=mesh,
        compiler_params=pltpu.CompilerParams(use_tc_tiling_on_sc=False),
        scratch_types=[
            pltpu.VMEM((_CHUNK // _GSLICE, _GSLICE), jnp.int32),
            pltpu.VMEM((_CHUNK // _GSLICE, _GSLICE), jnp.int32),
            pltpu.VMEM((_CHUNK, _C_IN), jnp.float32),
            pltpu.VMEM((_CHUNK, _C_IN), jnp.float32),
            pltpu.SemaphoreType.DMA,
            pltpu.SemaphoreType.DMA,
        ],
    )
    return f(node_features, src2d, dst2d)


def _dense_body(srcr, dstr, ef, w1, b1, w2, b2, out_ref):
    x = jnp.concatenate([srcr[...], dstr[...], ef[...]], axis=1)
    h = jnp.maximum(
        jnp.dot(x, w1[...], preferred_element_type=jnp.float32) + b1[...], 0.0)
    t = jnp.dot(h, w2[...], preferred_element_type=jnp.float32) + b2[...]
    mean = jnp.mean(t, axis=1, keepdims=True)
    var = jnp.mean((t - mean) ** 2, axis=1, keepdims=True)
    tn = (t - mean) * lax.rsqrt(var + 1e-5)
    out_ref[...] = ef[...] + jnp.maximum(tn, 0.0)


def _dense(srcr, dstr, ef, w1, b1, w2, b2, block_e=4000):
    grid = (_E // block_e,)
    bspec_e = pl.BlockSpec((block_e, _C_IN), lambda i: (i, 0))
    bspec_w = lambda a, b: pl.BlockSpec((a, b), lambda i: (0, 0))
    return pl.pallas_call(
        _dense_body,
        grid=grid,
        in_specs=[bspec_e, bspec_e, bspec_e,
                  bspec_w(3 * _C_IN, 2 * _C_HID), bspec_w(1, 2 * _C_HID),
                  bspec_w(2 * _C_HID, _C_IN), bspec_w(1, _C_IN)],
        out_specs=bspec_e,
        out_shape=jax.ShapeDtypeStruct((_E, _C_IN), jnp.float32),
    )(srcr, dstr, ef, w1, b1, w2, b2)


def kernel(node_features, edge_index, edge_features,
           W1a, b1a, W2a, b2a, W1b, b1b, W2b, b2b):
    pad = _E_PAD - _E
    src2d = jnp.concatenate(
        [edge_index[0], jnp.zeros((pad,), jnp.int32)]).reshape(_E_PAD // _GSLICE, _GSLICE)
    dst2d = jnp.concatenate(
        [edge_index[1], jnp.zeros((pad,), jnp.int32)]).reshape(_E_PAD // _GSLICE, _GSLICE)

    srcr, dstr = _sc_gather(node_features, src2d, dst2d)

    w1 = jnp.zeros((3 * _C_IN, 2 * _C_HID), jnp.float32)
    w1 = w1.at[0:_C_IN, 0:_C_HID].set(W1a.T)
    w1 = w1.at[_C_IN:2 * _C_IN, 0:_C_HID].set(W1a.T)
    w1 = w1.at[2 * _C_IN:, _C_HID:].set(W1b.T)
    b1 = jnp.concatenate([b1a, b1b]).reshape(1, 2 * _C_HID)
    w2 = jnp.concatenate([W2a.T, W2b.T], axis=0)
    b2 = (b2a + b2b).reshape(1, _C_IN)

    return _dense(srcr, dstr, edge_features, w1, b1, w2, b2)


# speedup vs baseline: 4.0534x; 2.0881x over previous
import functools

import jax
import jax.numpy as jnp
from jax import lax
from jax.experimental import pallas as pl
from jax.experimental.pallas import tpu as pltpu
from jax.experimental.pallas import tpu_sc as plsc

_E = 800000
_C_IN = 16
_C_HID = 64
_NW = 32
_CHUNK = 1024
_GSLICE = 128
_NSTEP = 25
_PER_W = _CHUNK * _NSTEP
_E_PAD = _PER_W * _NW
_TPAD = _CHUNK


def _gather_body(nf_hbm, src_hbm, dst_hbm, out_hbm,
                 sidx, didx, srows, drows, tbuf, sem_s, sem_d):
    wid = lax.axis_index("s") * 2 + lax.axis_index("c")
    row0 = wid * (_PER_W // _GSLICE)
    feat_off = jnp.arange(_C_IN, dtype=jnp.int32) * _CHUNK

    def step(c, carry):
        idx_row = row0 + c * (_CHUNK // _GSLICE)
        off = wid * _PER_W + c * _CHUNK
        pltpu.sync_copy(src_hbm.at[pl.ds(idx_row, _CHUNK // _GSLICE)], sidx)
        pltpu.sync_copy(dst_hbm.at[pl.ds(idx_row, _CHUNK // _GSLICE)], didx)
        copies = []
        for j in range(_CHUNK // _GSLICE):
            copies.append(pltpu.async_copy(
                nf_hbm.at[sidx.at[j]], srows.at[pl.ds(j * _GSLICE, _GSLICE)], sem_s))
            copies.append(pltpu.async_copy(
                nf_hbm.at[didx.at[j]], drows.at[pl.ds(j * _GSLICE, _GSLICE)], sem_d))
        for cp in copies:
            cp.wait()

        def add_t(g, carry2):
            for k in range(8):
                i = g * 8 + k
                s = srows[i] + drows[i]
                plsc.store_scatter(tbuf, [feat_off + i], s)
            return carry2

        lax.fori_loop(0, _CHUNK // 8, add_t, 0)
        for f in range(_C_IN):
            pltpu.sync_copy(tbuf.at[pl.ds(f * _CHUNK, _CHUNK)],
                            out_hbm.at[pl.ds(f * _E_PAD + off, _CHUNK)])
        return carry

    lax.fori_loop(0, _NSTEP, step, 0)


def _sc_gather(node_features, src2d, dst2d):
    mesh = plsc.VectorSubcoreMesh(core_axis_name="c", subcore_axis_name="s")
    f = pl.kernel(
        _gather_body,
        out_type=jax.ShapeDtypeStruct((_C_IN * _E_PAD,), jnp.float32),
        mesh=mesh,
        compiler_params=pltpu.CompilerParams(use_tc_tiling_on_sc=False,
                                             needs_layout_passes=False),
        scratch_types=[
            pltpu.VMEM((_CHUNK // _GSLICE, _GSLICE), jnp.int32),
            pltpu.VMEM((_CHUNK // _GSLICE, _GSLICE), jnp.int32),
            pltpu.VMEM((_CHUNK, _C_IN), jnp.float32),
            pltpu.VMEM((_CHUNK, _C_IN), jnp.float32),
            pltpu.VMEM((_C_IN * _CHUNK,), jnp.float32),
            pltpu.SemaphoreType.DMA,
            pltpu.SemaphoreType.DMA,
        ],
    )
    return f(node_features, src2d, dst2d)


def _dense_body(sum3, eft, w1, b1, w2, b2, out_ref):
    ns = sum3[...].reshape(_C_IN, -1)
    x = jnp.concatenate([ns, eft[...]], axis=0)
    h = jnp.maximum(
        jnp.dot(w1[...], x, preferred_element_type=jnp.float32) + b1[...], 0.0)
    t = jnp.dot(w2[...], h, preferred_element_type=jnp.float32) + b2[...]
    mean = jnp.mean(t, axis=0, keepdims=True)
    var = jnp.mean((t - mean) ** 2, axis=0, keepdims=True)
    tn = (t - mean) * lax.rsqrt(var + 1e-5)
    out_ref[...] = eft[...] + jnp.maximum(tn, 0.0)


def _dense(sum3, eft, w1, b1, w2, b2, block_e=4096):
    grid = (pl.cdiv(_E, block_e),)
    return pl.pallas_call(
        _dense_body,
        grid=grid,
        in_specs=[
            pl.BlockSpec((_C_IN, block_e // 128, 128), lambda i: (0, i, 0)),
            pl.BlockSpec((_C_IN, block_e), lambda i: (0, i)),
            pl.BlockSpec((2 * _C_HID, 2 * _C_IN), lambda i: (0, 0)),
            pl.BlockSpec((2 * _C_HID, 1), lambda i: (0, 0)),
            pl.BlockSpec((_C_IN, 2 * _C_HID), lambda i: (0, 0)),
            pl.BlockSpec((_C_IN, 1), lambda i: (0, 0)),
        ],
        out_specs=pl.BlockSpec((_C_IN, block_e), lambda i: (0, i)),
        out_shape=jax.ShapeDtypeStruct((_C_IN, _E), jnp.float32),
    )(sum3, eft, w1, b1, w2, b2)


def kernel(node_features, edge_index, edge_features,
           W1a, b1a, W2a, b2a, W1b, b1b, W2b, b2b):
    pad = _E_PAD - _E
    src2d = jnp.concatenate(
        [edge_index[0], jnp.zeros((pad,), jnp.int32)]).reshape(_E_PAD // _GSLICE, _GSLICE)
    dst2d = jnp.concatenate(
        [edge_index[1], jnp.zeros((pad,), jnp.int32)]).reshape(_E_PAD // _GSLICE, _GSLICE)

    sum_flat = _sc_gather(node_features, src2d, dst2d)
    sum3 = sum_flat.reshape(_C_IN, _E_PAD // 128, 128)

    w1 = jnp.zeros((2 * _C_HID, 2 * _C_IN), jnp.float32)
    w1 = w1.at[0:_C_HID, 0:_C_IN].set(W1a)
    w1 = w1.at[_C_HID:, _C_IN:].set(W1b)
    b1 = jnp.concatenate([b1a, b1b]).reshape(2 * _C_HID, 1)
    w2 = jnp.concatenate([W2a, W2b], axis=1)
    b2 = (b2a + b2b).reshape(_C_IN, 1)

    out_t = _dense(sum3, edge_features.T, w1, b1, w2, b2)
    return out_t.T


# speedup vs baseline: 5.8703x; 1.4482x over previous
import functools

import jax
import jax.numpy as jnp
from jax import lax
from jax.experimental import pallas as pl
from jax.experimental.pallas import tpu as pltpu
from jax.experimental.pallas import tpu_sc as plsc

_E = 800000
_C_IN = 16
_C_HID = 64
_NW = 32
_CHUNK = 640
_GSLICE = 128
_NR = _CHUNK // _GSLICE
_NSTEP = 40
_PER_W = _CHUNK * _NSTEP
_E_PAD = _PER_W * _NW


def _gather_body(nf_hbm, src_hbm, dst_hbm, out_hbm,
                 sidx0, sidx1, didx0, didx1, srows0, srows1, drows0, drows1,
                 tbuf0, tbuf1, si0, si1, sg0, sg1, sw0, sw1):
    SIDX, DIDX = [sidx0, sidx1], [didx0, didx1]
    SROWS, DROWS = [srows0, srows1], [drows0, drows1]
    TBUF, SI, SG, SW = [tbuf0, tbuf1], [si0, si1], [sg0, sg1], [sw0, sw1]

    wid = lax.axis_index("s") * 2 + lax.axis_index("c")
    row0 = wid * (_PER_W // _GSLICE)
    base = wid * _PER_W
    feat_off = jnp.arange(_C_IN, dtype=jnp.int32) * _CHUNK
    half = _NSTEP // 2

    def issue_idx(c, b):
        row = row0 + c * _NR
        pltpu.async_copy(src_hbm.at[pl.ds(row, _NR)], SIDX[b], SI[b])
        pltpu.async_copy(dst_hbm.at[pl.ds(row, _NR)], DIDX[b], SI[b])

    def wait_idx(b):
        pltpu.make_async_copy(src_hbm.at[pl.ds(0, _NR)], SIDX[b], SI[b]).wait()
        pltpu.make_async_copy(dst_hbm.at[pl.ds(0, _NR)], DIDX[b], SI[b]).wait()

    def issue_gather(b):
        for j in range(_NR):
            pltpu.async_copy(nf_hbm.at[SIDX[b].at[j]],
                             SROWS[b].at[pl.ds(j * _GSLICE, _GSLICE)], SG[b])
            pltpu.async_copy(nf_hbm.at[DIDX[b].at[j]],
                             DROWS[b].at[pl.ds(j * _GSLICE, _GSLICE)], SG[b])

    def wait_gather(b):
        pltpu.make_async_copy(nf_hbm.at[pl.ds(0, _CHUNK)], SROWS[b], SG[b]).wait()
        pltpu.make_async_copy(nf_hbm.at[pl.ds(0, _CHUNK)], DROWS[b], SG[b]).wait()

    def compute(b):
        def add_t(g2, carry2):
            for k in range(8):
                i = g2 * 8 + k
                s = SROWS[b][i] + DROWS[b][i]
                plsc.store_scatter(TBUF[b], [feat_off + i], s)
            return carry2
        lax.fori_loop(0, _CHUNK // 8, add_t, 0)

    def issue_write(c, b):
        off = base + c * _CHUNK
        for f in range(_C_IN):
            pltpu.async_copy(TBUF[b].at[pl.ds(f * _CHUNK, _CHUNK)],
                             out_hbm.at[pl.ds(f * _E_PAD + off, _CHUNK)], SW[b])

    def wait_write(b):
        pltpu.make_async_copy(TBUF[b], out_hbm.at[pl.ds(0, _C_IN * _CHUNK)],
                              SW[b]).wait()

    issue_idx(0, 0)
    wait_idx(0)
    issue_gather(0)
    issue_idx(1, 1)

    def iter_g(g, carry):
        for b in (0, 1):
            c = 2 * g + b
            nb = 1 - b

            def stage_next():
                wait_idx(nb)
                issue_gather(nb)
            if b == 0:
                stage_next()
            else:
                pl.when(g < half - 1)(stage_next)

            wait_gather(b)
            pl.when(g >= 1)(lambda: wait_write(b))
            compute(b)
            issue_write(c, b)
            pl.when(g < half - 1)(lambda: issue_idx(c + 2, b))
        return carry

    lax.fori_loop(0, half, iter_g, 0)
    wait_write(0)
    wait_write(1)


def _sc_gather(node_features, src2d, dst2d):
    mesh = plsc.VectorSubcoreMesh(core_axis_name="c", subcore_axis_name="s")
    f = pl.kernel(
        _gather_body,
        out_type=jax.ShapeDtypeStruct((_C_IN * _E_PAD,), jnp.float32),
        mesh=mesh,
        compiler_params=pltpu.CompilerParams(use_tc_tiling_on_sc=False,
                                             needs_layout_passes=False),
        scratch_types=(
            [pltpu.VMEM((_NR, _GSLICE), jnp.int32) for _ in range(4)]
            + [pltpu.VMEM((_CHUNK, _C_IN), jnp.float32) for _ in range(4)]
            + [pltpu.VMEM((_C_IN * _CHUNK,), jnp.float32) for _ in range(2)]
            + [pltpu.SemaphoreType.DMA for _ in range(6)]
        ),
    )
    return f(node_features, src2d, dst2d)


def _dense_body(sum3, eft, w1, b1, w2, b2, out_ref):
    ns = sum3[...].reshape(_C_IN, -1)
    x = jnp.concatenate([ns, eft[...]], axis=0)
    h = jnp.maximum(
        jnp.dot(w1[...], x, preferred_element_type=jnp.float32) + b1[...], 0.0)
    t = jnp.dot(w2[...], h, preferred_element_type=jnp.float32) + b2[...]
    mean = jnp.mean(t, axis=0, keepdims=True)
    var = jnp.mean((t - mean) ** 2, axis=0, keepdims=True)
    tn = (t - mean) * lax.rsqrt(var + 1e-5)
    out_ref[...] = eft[...] + jnp.maximum(tn, 0.0)


def _dense(sum3, eft, w1, b1, w2, b2, block_e=4096):
    grid = (pl.cdiv(_E, block_e),)
    return pl.pallas_call(
        _dense_body,
        grid=grid,
        in_specs=[
            pl.BlockSpec((_C_IN, block_e // 128, 128), lambda i: (0, i, 0)),
            pl.BlockSpec((_C_IN, block_e), lambda i: (0, i)),
            pl.BlockSpec((2 * _C_HID, 2 * _C_IN), lambda i: (0, 0)),
            pl.BlockSpec((2 * _C_HID, 1), lambda i: (0, 0)),
            pl.BlockSpec((_C_IN, 2 * _C_HID), lambda i: (0, 0)),
            pl.BlockSpec((_C_IN, 1), lambda i: (0, 0)),
        ],
        out_specs=pl.BlockSpec((_C_IN, block_e), lambda i: (0, i)),
        out_shape=jax.ShapeDtypeStruct((_C_IN, _E), jnp.float32),
    )(sum3, eft, w1, b1, w2, b2)


def kernel(node_features, edge_index, edge_features,
           W1a, b1a, W2a, b2a, W1b, b1b, W2b, b2b):
    pad = _E_PAD - _E
    src2d = jnp.concatenate(
        [edge_index[0], jnp.zeros((pad,), jnp.int32)]).reshape(_E_PAD // _GSLICE, _GSLICE)
    dst2d = jnp.concatenate(
        [edge_index[1], jnp.zeros((pad,), jnp.int32)]).reshape(_E_PAD // _GSLICE, _GSLICE)

    sum_flat = _sc_gather(node_features, src2d, dst2d)
    sum3 = sum_flat.reshape(_C_IN, _E_PAD // 128, 128)

    w1 = jnp.zeros((2 * _C_HID, 2 * _C_IN), jnp.float32)
    w1 = w1.at[0:_C_HID, 0:_C_IN].set(W1a)
    w1 = w1.at[_C_HID:, _C_IN:].set(W1b)
    b1 = jnp.concatenate([b1a, b1b]).reshape(2 * _C_HID, 1)
    w2 = jnp.concatenate([W2a, W2b], axis=1)
    b2 = (b2a + b2b).reshape(_C_IN, 1)

    out_t = _dense(sum3, edge_features.T, w1, b1, w2, b2)
    return out_t.T


# speedup vs baseline: 6.0388x; 1.0287x over previous
import functools

import jax
import jax.numpy as jnp
from jax import lax
from jax.experimental import pallas as pl
from jax.experimental.pallas import tpu as pltpu
from jax.experimental.pallas import tpu_sc as plsc

_E = 800000
_C_IN = 16
_C_HID = 64
_NW = 32
_CHUNK = 1280
_GSLICE = 128
_NR = _CHUNK // _GSLICE
_NSTEP = 20
_PER_W = _CHUNK * _NSTEP
_E_PAD = _PER_W * _NW


def _gather_body(nf_hbm, src_hbm, dst_hbm, out_hbm,
                 sidx0, sidx1, didx0, didx1, srows0, srows1, drows0, drows1,
                 tbuf0, tbuf1, si0, si1, sg0, sg1, sw0, sw1):
    SIDX, DIDX = [sidx0, sidx1], [didx0, didx1]
    SROWS, DROWS = [srows0, srows1], [drows0, drows1]
    TBUF, SI, SG, SW = [tbuf0, tbuf1], [si0, si1], [sg0, sg1], [sw0, sw1]

    wid = lax.axis_index("s") * 2 + lax.axis_index("c")
    row0 = wid * (_PER_W // _GSLICE)
    base = wid * _PER_W
    feat_off = jnp.arange(_C_IN, dtype=jnp.int32) * _CHUNK
    half = _NSTEP // 2

    def issue_idx(c, b):
        row = row0 + c * _NR
        pltpu.async_copy(src_hbm.at[pl.ds(row, _NR)], SIDX[b], SI[b])
        pltpu.async_copy(dst_hbm.at[pl.ds(row, _NR)], DIDX[b], SI[b])

    def wait_idx(b):
        pltpu.make_async_copy(src_hbm.at[pl.ds(0, _NR)], SIDX[b], SI[b]).wait()
        pltpu.make_async_copy(dst_hbm.at[pl.ds(0, _NR)], DIDX[b], SI[b]).wait()

    def issue_gather(b):
        for j in range(_NR):
            pltpu.async_copy(nf_hbm.at[SIDX[b].at[j]],
                             SROWS[b].at[pl.ds(j * _GSLICE, _GSLICE)], SG[b])
            pltpu.async_copy(nf_hbm.at[DIDX[b].at[j]],
                             DROWS[b].at[pl.ds(j * _GSLICE, _GSLICE)], SG[b])

    def wait_gather(b):
        pltpu.make_async_copy(nf_hbm.at[pl.ds(0, _CHUNK)], SROWS[b], SG[b]).wait()
        pltpu.make_async_copy(nf_hbm.at[pl.ds(0, _CHUNK)], DROWS[b], SG[b]).wait()

    def compute(b):
        @plsc.parallel_loop(0, _CHUNK, 1, unroll=8)
        def _(i):
            s = SROWS[b][i] + DROWS[b][i]
            plsc.store_scatter(TBUF[b], [feat_off + i], s)

    def issue_write(c, b):
        off = base + c * _CHUNK
        for f in range(_C_IN):
            pltpu.async_copy(TBUF[b].at[pl.ds(f * _CHUNK, _CHUNK)],
                             out_hbm.at[pl.ds(f * _E_PAD + off, _CHUNK)], SW[b])

    def wait_write(b):
        pltpu.make_async_copy(TBUF[b], out_hbm.at[pl.ds(0, _C_IN * _CHUNK)],
                              SW[b]).wait()

    issue_idx(0, 0)
    wait_idx(0)
    issue_gather(0)
    issue_idx(1, 1)

    def iter_g(g, carry):
        for b in (0, 1):
            c = 2 * g + b
            nb = 1 - b

            def stage_next():
                wait_idx(nb)
                issue_gather(nb)
            if b == 0:
                stage_next()
            else:
                pl.when(g < half - 1)(stage_next)

            wait_gather(b)
            pl.when(g >= 1)(lambda: wait_write(b))
            compute(b)
            issue_write(c, b)
            pl.when(g < half - 1)(lambda: issue_idx(c + 2, b))
        return carry

    lax.fori_loop(0, half, iter_g, 0)
    wait_write(0)
    wait_write(1)


def _sc_gather(node_features, src2d, dst2d):
    mesh = plsc.VectorSubcoreMesh(core_axis_name="c", subcore_axis_name="s")
    f = pl.kernel(
        _gather_body,
        out_type=jax.ShapeDtypeStruct((_C_IN * _E_PAD,), jnp.float32),
        mesh=mesh,
        compiler_params=pltpu.CompilerParams(use_tc_tiling_on_sc=False,
                                             needs_layout_passes=False),
        scratch_types=(
            [pltpu.VMEM((_NR, _GSLICE), jnp.int32) for _ in range(4)]
            + [pltpu.VMEM((_CHUNK, _C_IN), jnp.float32) for _ in range(4)]
            + [pltpu.VMEM((_C_IN * _CHUNK,), jnp.float32) for _ in range(2)]
            + [pltpu.SemaphoreType.DMA for _ in range(6)]
        ),
    )
    return f(node_features, src2d, dst2d)


def _dense_body(sum3, eft, w1, b1, w2, b2, out_ref):
    ns = sum3[...].reshape(_C_IN, -1)
    x = jnp.concatenate([ns, eft[...]], axis=0)
    h = jnp.maximum(
        jnp.dot(w1[...], x, preferred_element_type=jnp.float32) + b1[...], 0.0)
    t = jnp.dot(w2[...], h, preferred_element_type=jnp.float32) + b2[...]
    mean = jnp.mean(t, axis=0, keepdims=True)
    var = jnp.mean((t - mean) ** 2, axis=0, keepdims=True)
    tn = (t - mean) * lax.rsqrt(var + 1e-5)
    out_ref[...] = eft[...] + jnp.maximum(tn, 0.0)


def _dense(sum3, eft, w1, b1, w2, b2, block_e=4096):
    grid = (pl.cdiv(_E, block_e),)
    return pl.pallas_call(
        _dense_body,
        grid=grid,
        in_specs=[
            pl.BlockSpec((_C_IN, block_e // 128, 128), lambda i: (0, i, 0)),
            pl.BlockSpec((_C_IN, block_e), lambda i: (0, i)),
            pl.BlockSpec((2 * _C_HID, 2 * _C_IN), lambda i: (0, 0)),
            pl.BlockSpec((2 * _C_HID, 1), lambda i: (0, 0)),
            pl.BlockSpec((_C_IN, 2 * _C_HID), lambda i: (0, 0)),
            pl.BlockSpec((_C_IN, 1), lambda i: (0, 0)),
        ],
        out_specs=pl.BlockSpec((_C_IN, block_e), lambda i: (0, i)),
        out_shape=jax.ShapeDtypeStruct((_C_IN, _E), jnp.float32),
    )(sum3, eft, w1, b1, w2, b2)


def kernel(node_features, edge_index, edge_features,
           W1a, b1a, W2a, b2a, W1b, b1b, W2b, b2b):
    pad = _E_PAD - _E
    src2d = jnp.concatenate(
        [edge_index[0], jnp.zeros((pad,), jnp.int32)]).reshape(_E_PAD // _GSLICE, _GSLICE)
    dst2d = jnp.concatenate(
        [edge_index[1], jnp.zeros((pad,), jnp.int32)]).reshape(_E_PAD // _GSLICE, _GSLICE)

    sum_flat = _sc_gather(node_features, src2d, dst2d)
    sum3 = sum_flat.reshape(_C_IN, _E_PAD // 128, 128)

    w1 = jnp.zeros((2 * _C_HID, 2 * _C_IN), jnp.float32)
    w1 = w1.at[0:_C_HID, 0:_C_IN].set(W1a)
    w1 = w1.at[_C_HID:, _C_IN:].set(W1b)
    b1 = jnp.concatenate([b1a, b1b]).reshape(2 * _C_HID, 1)
    w2 = jnp.concatenate([W2a, W2b], axis=1)
    b2 = (b2a + b2b).reshape(_C_IN, 1)

    out_t = _dense(sum3, edge_features.T, w1, b1, w2, b2)
    return out_t.T


# speedup vs baseline: 6.5830x; 1.0901x over previous
import functools

import jax
import jax.numpy as jnp
from jax import lax
from jax.experimental import pallas as pl
from jax.experimental.pallas import tpu as pltpu
from jax.experimental.pallas import tpu_sc as plsc

_E = 800000
_C_IN = 16
_C_HID = 64
_NW = 32
_CHUNK = 1280
_GSLICE = 128
_NR = _CHUNK // _GSLICE
_NSTEP = 20
_PER_W = _CHUNK * _NSTEP
_E_PAD = _PER_W * _NW


def _gather_body(nf_hbm, src_hbm, dst_hbm, out_hbm,
                 sidx0, sidx1, didx0, didx1, srows0, srows1, drows0, drows1,
                 tbuf0, tbuf1, si0, si1, sg0, sg1, sw0, sw1,
                 *, h, nstep, e_span):
    SIDX, DIDX = [sidx0, sidx1], [didx0, didx1]
    SROWS, DROWS = [srows0, srows1], [drows0, drows1]
    TBUF, SI, SG, SW = [tbuf0, tbuf1], [si0, si1], [sg0, sg1], [sw0, sw1]

    per_w = _CHUNK * nstep
    wid = lax.axis_index("s") * 2 + lax.axis_index("c")
    row0 = (h * e_span + wid * per_w) // _GSLICE
    base = wid * per_w
    feat_off = jnp.arange(_C_IN, dtype=jnp.int32) * _CHUNK
    half = nstep // 2

    def issue_idx(c, b):
        row = row0 + c * _NR
        pltpu.async_copy(src_hbm.at[pl.ds(row, _NR)], SIDX[b], SI[b])
        pltpu.async_copy(dst_hbm.at[pl.ds(row, _NR)], DIDX[b], SI[b])

    def wait_idx(b):
        pltpu.make_async_copy(src_hbm.at[pl.ds(0, _NR)], SIDX[b], SI[b]).wait()
        pltpu.make_async_copy(dst_hbm.at[pl.ds(0, _NR)], DIDX[b], SI[b]).wait()

    def issue_gather(b):
        for j in range(_NR):
            pltpu.async_copy(nf_hbm.at[SIDX[b].at[j]],
                             SROWS[b].at[pl.ds(j * _GSLICE, _GSLICE)], SG[b])
            pltpu.async_copy(nf_hbm.at[DIDX[b].at[j]],
                             DROWS[b].at[pl.ds(j * _GSLICE, _GSLICE)], SG[b])

    def wait_gather(b):
        pltpu.make_async_copy(nf_hbm.at[pl.ds(0, _CHUNK)], SROWS[b], SG[b]).wait()
        pltpu.make_async_copy(nf_hbm.at[pl.ds(0, _CHUNK)], DROWS[b], SG[b]).wait()

    def compute(b):
        @plsc.parallel_loop(0, _CHUNK, 1, unroll=8)
        def _(i):
            s = SROWS[b][i] + DROWS[b][i]
            plsc.store_scatter(TBUF[b], [feat_off + i], s)

    def issue_write(c, b):
        off = base + c * _CHUNK
        for f in range(_C_IN):
            pltpu.async_copy(TBUF[b].at[pl.ds(f * _CHUNK, _CHUNK)],
                             out_hbm.at[pl.ds(f * e_span + off, _CHUNK)], SW[b])

    def wait_write(b):
        pltpu.make_async_copy(TBUF[b], out_hbm.at[pl.ds(0, _C_IN * _CHUNK)],
                              SW[b]).wait()

    issue_idx(0, 0)
    wait_idx(0)
    issue_gather(0)
    issue_idx(1, 1)

    def iter_g(g, carry):
        for b in (0, 1):
            c = 2 * g + b
            nb = 1 - b

            def stage_next():
                wait_idx(nb)
                issue_gather(nb)
            if b == 0:
                stage_next()
            else:
                pl.when(g < half - 1)(stage_next)

            wait_gather(b)
            pl.when(g >= 1)(lambda: wait_write(b))
            compute(b)
            issue_write(c, b)
            pl.when(g < half - 1)(lambda: issue_idx(c + 2, b))
        return carry

    lax.fori_loop(0, half, iter_g, 0)
    wait_write(0)
    wait_write(1)


def _sc_gather(node_features, src2d, dst2d, h, nsplit):
    e_span = _E_PAD // nsplit
    nstep = _NSTEP // nsplit
    mesh = plsc.VectorSubcoreMesh(core_axis_name="c", subcore_axis_name="s")
    f = pl.kernel(
        functools.partial(_gather_body, h=h, nstep=nstep, e_span=e_span),
        out_type=jax.ShapeDtypeStruct((_C_IN * e_span,), jnp.float32),
        mesh=mesh,
        compiler_params=pltpu.CompilerParams(use_tc_tiling_on_sc=False,
                                             needs_layout_passes=False),
        scratch_types=(
            [pltpu.VMEM((_NR, _GSLICE), jnp.int32) for _ in range(4)]
            + [pltpu.VMEM((_CHUNK, _C_IN), jnp.float32) for _ in range(4)]
            + [pltpu.VMEM((_C_IN * _CHUNK,), jnp.float32) for _ in range(2)]
            + [pltpu.SemaphoreType.DMA for _ in range(6)]
        ),
    )
    return f(node_features, src2d, dst2d)


def _dense_body(sum3, eft, w1, b1, w2, b2, out_ref):
    ns = sum3[...].reshape(_C_IN, -1)
    x = jnp.concatenate([ns, eft[...]], axis=0)
    h = jnp.maximum(
        jnp.dot(w1[...], x, preferred_element_type=jnp.float32) + b1[...], 0.0)
    t = jnp.dot(w2[...], h, preferred_element_type=jnp.float32) + b2[...]
    mean = jnp.mean(t, axis=0, keepdims=True)
    var = jnp.mean((t - mean) ** 2, axis=0, keepdims=True)
    tn = (t - mean) * lax.rsqrt(var + 1e-5)
    out_ref[...] = eft[...] + jnp.maximum(tn, 0.0)


def _dense_half(sum3, eft, w1, b1, w2, b2, prev, block0, nblk, block_e=4096):
    body = _dense_body
    in_specs = [
        pl.BlockSpec((_C_IN, block_e // 128, 128), lambda i: (0, i, 0)),
        pl.BlockSpec((_C_IN, block_e), lambda i: (0, i + block0)),
        pl.BlockSpec((2 * _C_HID, 2 * _C_IN), lambda i: (0, 0)),
        pl.BlockSpec((2 * _C_HID, 1), lambda i: (0, 0)),
        pl.BlockSpec((_C_IN, 2 * _C_HID), lambda i: (0, 0)),
        pl.BlockSpec((_C_IN, 1), lambda i: (0, 0)),
    ]
    args = [sum3, eft, w1, b1, w2, b2]
    kwargs = {}
    if prev is not None:
        def body(sum3, eft, w1, b1, w2, b2, prev_ref, out_ref):
            _dense_body(sum3, eft, w1, b1, w2, b2, out_ref)
        in_specs.append(pl.BlockSpec(memory_space=pl.ANY))
        args.append(prev)
        kwargs["input_output_aliases"] = {6: 0}
    return pl.pallas_call(
        body,
        grid=(nblk,),
        in_specs=in_specs,
        out_specs=pl.BlockSpec((_C_IN, block_e), lambda i: (0, i + block0)),
        out_shape=jax.ShapeDtypeStruct((_C_IN, _E), jnp.float32),
        **kwargs,
    )(*args)


def kernel(node_features, edge_index, edge_features,
           W1a, b1a, W2a, b2a, W1b, b1b, W2b, b2b):
    pad = _E_PAD - _E
    src2d = jnp.concatenate(
        [edge_index[0], jnp.zeros((pad,), jnp.int32)]).reshape(_E_PAD // _GSLICE, _GSLICE)
    dst2d = jnp.concatenate(
        [edge_index[1], jnp.zeros((pad,), jnp.int32)]).reshape(_E_PAD // _GSLICE, _GSLICE)

    nsplit = 2
    e_span = _E_PAD // nsplit
    sums = [_sc_gather(node_features, src2d, dst2d, h, nsplit)
            .reshape(_C_IN, e_span // 128, 128) for h in range(nsplit)]

    w1 = jnp.zeros((2 * _C_HID, 2 * _C_IN), jnp.float32)
    w1 = w1.at[0:_C_HID, 0:_C_IN].set(W1a)
    w1 = w1.at[_C_HID:, _C_IN:].set(W1b)
    b1 = jnp.concatenate([b1a, b1b]).reshape(2 * _C_HID, 1)
    w2 = jnp.concatenate([W2a, W2b], axis=1)
    b2 = (b2a + b2b).reshape(_C_IN, 1)

    eft = edge_features.T
    block_e = 4096
    blocks_per_half = e_span // block_e
    total_blocks = pl.cdiv(_E, block_e)
    out = None
    for h in range(nsplit):
        block0 = h * blocks_per_half
        nblk = min(blocks_per_half, total_blocks - block0)
        out = _dense_half(sums[h], eft, w1, b1, w2, b2, out,
                          block0, nblk, block_e)
    return out.T


# speedup vs baseline: 8.8152x; 1.3391x over previous
import functools

import jax
import jax.numpy as jnp
from jax import lax
from jax.experimental import pallas as pl
from jax.experimental.pallas import tpu as pltpu
from jax.experimental.pallas import tpu_sc as plsc

_E = 800000
_C_IN = 16
_C_HID = 64
_NW = 32
_CHUNK = 640
_GSLICE = 128
_NR = _CHUNK // _GSLICE
_NSTEP = 40
_PER_W = _CHUNK * _NSTEP
_E_PAD = _PER_W * _NW


def _gather_body(nf_hbm, src_hbm, dst_hbm, out_hbm,
                 sidx0, sidx1, didx0, didx1, srows0, srows1, drows0, drows1,
                 tbuf0, tbuf1, shared_nf, si0, si1, sg0, sg1, sw0, sw1,
                 *, h, nstep, e_span):
    SIDX, DIDX = [sidx0, sidx1], [didx0, didx1]
    SROWS, DROWS = [srows0, srows1], [drows0, drows1]
    TBUF, SI, SG, SW = [tbuf0, tbuf1], [si0, si1], [sg0, sg1], [sw0, sw1]

    per_w = _CHUNK * nstep
    wid = lax.axis_index("s") * 2 + lax.axis_index("c")
    row0 = (h * e_span + wid * per_w) // _GSLICE
    base = wid * per_w
    feat_off = jnp.arange(_C_IN, dtype=jnp.int32) * _CHUNK
    half = nstep // 2

    def issue_idx(c, b):
        row = row0 + c * _NR
        pltpu.async_copy(src_hbm.at[pl.ds(row, _NR)], SIDX[b], SI[b])
        pltpu.async_copy(dst_hbm.at[pl.ds(row, _NR)], DIDX[b], SI[b])

    def wait_idx(b):
        pltpu.make_async_copy(src_hbm.at[pl.ds(0, _NR)], SIDX[b], SI[b]).wait()
        pltpu.make_async_copy(dst_hbm.at[pl.ds(0, _NR)], DIDX[b], SI[b]).wait()

    def issue_gather(b):
        for j in range(_NR):
            pltpu.async_copy(shared_nf.at[SIDX[b].at[j]],
                             SROWS[b].at[pl.ds(j * _GSLICE, _GSLICE)], SG[b])
            pltpu.async_copy(shared_nf.at[DIDX[b].at[j]],
                             DROWS[b].at[pl.ds(j * _GSLICE, _GSLICE)], SG[b])

    def wait_gather(b):
        pltpu.make_async_copy(nf_hbm.at[pl.ds(0, _CHUNK)], SROWS[b], SG[b]).wait()
        pltpu.make_async_copy(nf_hbm.at[pl.ds(0, _CHUNK)], DROWS[b], SG[b]).wait()

    def compute(b):
        @plsc.parallel_loop(0, _CHUNK, 1, unroll=8)
        def _(i):
            s = SROWS[b][i] + DROWS[b][i]
            plsc.store_scatter(TBUF[b], [feat_off + i], s)

    def issue_write(c, b):
        off = base + c * _CHUNK
        for f in range(_C_IN):
            pltpu.async_copy(TBUF[b].at[pl.ds(f * _CHUNK, _CHUNK)],
                             out_hbm.at[pl.ds(f * e_span + off, _CHUNK)], SW[b])

    def wait_write(b):
        pltpu.make_async_copy(TBUF[b], out_hbm.at[pl.ds(0, _C_IN * _CHUNK)],
                              SW[b]).wait()

    @pl.when(lax.axis_index("s") == 0)
    def _():
        pltpu.sync_copy(nf_hbm, shared_nf)
    plsc.subcore_barrier()

    issue_idx(0, 0)
    wait_idx(0)
    issue_gather(0)
    issue_idx(1, 1)

    def iter_g(g, carry):
        for b in (0, 1):
            c = 2 * g + b
            nb = 1 - b

            def stage_next():
                wait_idx(nb)
                issue_gather(nb)
            if b == 0:
                stage_next()
            else:
                pl.when(g < half - 1)(stage_next)

            wait_gather(b)
            pl.when(g >= 1)(lambda: wait_write(b))
            compute(b)
            issue_write(c, b)
            pl.when(g < half - 1)(lambda: issue_idx(c + 2, b))
        return carry

    lax.fori_loop(0, half, iter_g, 0)
    wait_write(0)
    wait_write(1)


def _sc_gather(node_features, src2d, dst2d, h, nsplit):
    e_span = _E_PAD // nsplit
    nstep = _NSTEP // nsplit
    mesh = plsc.VectorSubcoreMesh(core_axis_name="c", subcore_axis_name="s")
    f = pl.kernel(
        functools.partial(_gather_body, h=h, nstep=nstep, e_span=e_span),
        out_type=jax.ShapeDtypeStruct((_C_IN * e_span,), jnp.float32),
        mesh=mesh,
        compiler_params=pltpu.CompilerParams(use_tc_tiling_on_sc=False,
                                             needs_layout_passes=False),
        scratch_types=(
            [pltpu.VMEM((_NR, _GSLICE), jnp.int32) for _ in range(4)]
            + [pltpu.VMEM((_CHUNK, _C_IN), jnp.float32) for _ in range(4)]
            + [pltpu.VMEM((_C_IN * _CHUNK,), jnp.float32) for _ in range(2)]
            + [pltpu.VMEM_SHARED((50000, _C_IN), jnp.float32)]
            + [pltpu.SemaphoreType.DMA for _ in range(6)]
        ),
    )
    return f(node_features, src2d, dst2d)


def _dense_body(sum3, eft, w1, b1, w2, b2, out_ref):
    ns = sum3[...].reshape(_C_IN, -1)
    x = jnp.concatenate([ns, eft[...]], axis=0)
    h = jnp.maximum(
        jnp.dot(w1[...], x, preferred_element_type=jnp.float32) + b1[...], 0.0)
    t = jnp.dot(w2[...], h, preferred_element_type=jnp.float32) + b2[...]
    mean = jnp.mean(t, axis=0, keepdims=True)
    var = jnp.mean((t - mean) ** 2, axis=0, keepdims=True)
    tn = (t - mean) * lax.rsqrt(var + 1e-5)
    out_ref[...] = eft[...] + jnp.maximum(tn, 0.0)


def _dense_half(sum3, eft, w1, b1, w2, b2, prev, block0, nblk, block_e=4096):
    body = _dense_body
    in_specs = [
        pl.BlockSpec((_C_IN, block_e // 128, 128), lambda i: (0, i, 0)),
        pl.BlockSpec((_C_IN, block_e), lambda i: (0, i + block0)),
        pl.BlockSpec((2 * _C_HID, 2 * _C_IN), lambda i: (0, 0)),
        pl.BlockSpec((2 * _C_HID, 1), lambda i: (0, 0)),
        pl.BlockSpec((_C_IN, 2 * _C_HID), lambda i: (0, 0)),
        pl.BlockSpec((_C_IN, 1), lambda i: (0, 0)),
    ]
    args = [sum3, eft, w1, b1, w2, b2]
    kwargs = {}
    if prev is not None:
        def body(sum3, eft, w1, b1, w2, b2, prev_ref, out_ref):
            _dense_body(sum3, eft, w1, b1, w2, b2, out_ref)
        in_specs.append(pl.BlockSpec(memory_space=pl.ANY))
        args.append(prev)
        kwargs["input_output_aliases"] = {6: 0}
    return pl.pallas_call(
        body,
        grid=(nblk,),
        in_specs=in_specs,
        out_specs=pl.BlockSpec((_C_IN, block_e), lambda i: (0, i + block0)),
        out_shape=jax.ShapeDtypeStruct((_C_IN, _E), jnp.float32),
        **kwargs,
    )(*args)


def kernel(node_features, edge_index, edge_features,
           W1a, b1a, W2a, b2a, W1b, b1b, W2b, b2b):
    pad = _E_PAD - _E
    src2d = jnp.concatenate(
        [edge_index[0], jnp.zeros((pad,), jnp.int32)]).reshape(_E_PAD // _GSLICE, _GSLICE)
    dst2d = jnp.concatenate(
        [edge_index[1], jnp.zeros((pad,), jnp.int32)]).reshape(_E_PAD // _GSLICE, _GSLICE)

    nsplit = 2
    e_span = _E_PAD // nsplit
    sums = [_sc_gather(node_features, src2d, dst2d, h, nsplit)
            .reshape(_C_IN, e_span // 128, 128) for h in range(nsplit)]

    w1 = jnp.zeros((2 * _C_HID, 2 * _C_IN), jnp.float32)
    w1 = w1.at[0:_C_HID, 0:_C_IN].set(W1a)
    w1 = w1.at[_C_HID:, _C_IN:].set(W1b)
    b1 = jnp.concatenate([b1a, b1b]).reshape(2 * _C_HID, 1)
    w2 = jnp.concatenate([W2a, W2b], axis=1)
    b2 = (b2a + b2b).reshape(_C_IN, 1)

    eft = edge_features.T
    block_e = 4096
    blocks_per_half = e_span // block_e
    total_blocks = pl.cdiv(_E, block_e)
    out = None
    for h in range(nsplit):
        block0 = h * blocks_per_half
        nblk = min(blocks_per_half, total_blocks - block0)
        out = _dense_half(sums[h], eft, w1, b1, w2, b2, out,
                          block0, nblk, block_e)
    return out.T


# speedup vs baseline: 9.3948x; 1.0657x over previous
import functools

import jax
import jax.numpy as jnp
from jax import lax
from jax.experimental import pallas as pl
from jax.experimental.pallas import tpu as pltpu
from jax.experimental.pallas import tpu_sc as plsc

_E = 800000
_C_IN = 16
_C_HID = 64
_NW = 32
_CHUNK = 640
_GSLICE = 128
_NR = _CHUNK // _GSLICE
_NSTEP = 40
_PER_W = _CHUNK * _NSTEP
_E_PAD = _PER_W * _NW


def _gather_body(nf_hbm, src_hbm, dst_hbm, out_hbm,
                 sidx0, sidx1, didx0, didx1, srows0, srows1, drows0, drows1,
                 tbuf0, tbuf1, shared_nf, si0, si1, sg0, sg1, sw0, sw1,
                 *, h, nstep, e_span):
    SIDX, DIDX = [sidx0, sidx1], [didx0, didx1]
    SROWS, DROWS = [srows0, srows1], [drows0, drows1]
    TBUF, SI, SG, SW = [tbuf0, tbuf1], [si0, si1], [sg0, sg1], [sw0, sw1]

    per_w = _CHUNK * nstep
    wid = lax.axis_index("s") * 2 + lax.axis_index("c")
    row0 = (h * e_span + wid * per_w) // _GSLICE
    base = wid * per_w
    feat_off = jnp.arange(_C_IN, dtype=jnp.int32) * _CHUNK
    half = nstep // 2

    def issue_idx(c, b):
        row = row0 + c * _NR
        pltpu.async_copy(src_hbm.at[pl.ds(row, _NR)], SIDX[b], SI[b])
        pltpu.async_copy(dst_hbm.at[pl.ds(row, _NR)], DIDX[b], SI[b])

    def wait_idx(b):
        pltpu.make_async_copy(src_hbm.at[pl.ds(0, _NR)], SIDX[b], SI[b]).wait()
        pltpu.make_async_copy(dst_hbm.at[pl.ds(0, _NR)], DIDX[b], SI[b]).wait()

    def issue_gather(b):
        for j in range(_NR):
            pltpu.async_copy(shared_nf.at[SIDX[b].at[j]],
                             SROWS[b].at[pl.ds(j * _GSLICE, _GSLICE)], SG[b])
            pltpu.async_copy(shared_nf.at[DIDX[b].at[j]],
                             DROWS[b].at[pl.ds(j * _GSLICE, _GSLICE)], SG[b])

    def wait_gather(b):
        pltpu.make_async_copy(nf_hbm.at[pl.ds(0, _CHUNK)], SROWS[b], SG[b]).wait()
        pltpu.make_async_copy(nf_hbm.at[pl.ds(0, _CHUNK)], DROWS[b], SG[b]).wait()

    def compute(b):
        @plsc.parallel_loop(0, _CHUNK, 1, unroll=8)
        def _(i):
            s = SROWS[b][i] + DROWS[b][i]
            plsc.store_scatter(TBUF[b], [feat_off + i], s)

    def issue_write(c, b):
        off = base + c * _CHUNK
        for f in range(_C_IN):
            pltpu.async_copy(TBUF[b].at[pl.ds(f * _CHUNK, _CHUNK)],
                             out_hbm.at[pl.ds(f * e_span + off, _CHUNK)], SW[b])

    def wait_write(b):
        pltpu.make_async_copy(TBUF[b], out_hbm.at[pl.ds(0, _C_IN * _CHUNK)],
                              SW[b]).wait()

    @pl.when(lax.axis_index("s") == 0)
    def _():
        pltpu.sync_copy(nf_hbm, shared_nf)
    plsc.subcore_barrier()

    issue_idx(0, 0)
    wait_idx(0)
    issue_gather(0)
    issue_idx(1, 1)

    def iter_g(g, carry):
        for b in (0, 1):
            c = 2 * g + b
            nb = 1 - b

            def stage_next():
                wait_idx(nb)
                issue_gather(nb)
            if b == 0:
                stage_next()
            else:
                pl.when(g < half - 1)(stage_next)

            wait_gather(b)
            pl.when(g >= 1)(lambda: wait_write(b))
            compute(b)
            issue_write(c, b)
            pl.when(g < half - 1)(lambda: issue_idx(c + 2, b))
        return carry

    lax.fori_loop(0, half, iter_g, 0)
    wait_write(0)
    wait_write(1)


def _sc_gather(node_features, src2d, dst2d, h, nsplit):
    e_span = _E_PAD // nsplit
    nstep = _NSTEP // nsplit
    mesh = plsc.VectorSubcoreMesh(core_axis_name="c", subcore_axis_name="s")
    f = pl.kernel(
        functools.partial(_gather_body, h=h, nstep=nstep, e_span=e_span),
        out_type=jax.ShapeDtypeStruct((_C_IN * e_span,), jnp.float32),
        mesh=mesh,
        compiler_params=pltpu.CompilerParams(use_tc_tiling_on_sc=False,
                                             needs_layout_passes=False),
        scratch_types=(
            [pltpu.VMEM((_NR, _GSLICE), jnp.int32) for _ in range(4)]
            + [pltpu.VMEM((_CHUNK, _C_IN), jnp.float32) for _ in range(4)]
            + [pltpu.VMEM((_C_IN * _CHUNK,), jnp.float32) for _ in range(2)]
            + [pltpu.VMEM_SHARED((50000, _C_IN), jnp.float32)]
            + [pltpu.SemaphoreType.DMA for _ in range(6)]
        ),
    )
    return f(node_features, src2d, dst2d)


def _dense_body(sum3, eft, w1, b1, w2, b2, out_ref):
    ns = sum3[...].reshape(_C_IN, -1)
    x = jnp.concatenate([ns, eft[...]], axis=0)
    h = jnp.maximum(
        jnp.dot(w1[...], x, preferred_element_type=jnp.float32) + b1[...], 0.0)
    t = jnp.dot(w2[...], h, preferred_element_type=jnp.float32) + b2[...]
    mean = jnp.mean(t, axis=0, keepdims=True)
    var = jnp.mean((t - mean) ** 2, axis=0, keepdims=True)
    tn = (t - mean) * lax.rsqrt(var + 1e-5)
    out_ref[...] = eft[...] + jnp.maximum(tn, 0.0)


def _dense_half(sum3, eft, w1, b1, w2, b2, prev, block0, nblk, block_e=4096):
    body = _dense_body
    in_specs = [
        pl.BlockSpec((_C_IN, block_e // 128, 128), lambda i: (0, i, 0)),
        pl.BlockSpec((_C_IN, block_e), lambda i: (0, i + block0)),
        pl.BlockSpec((2 * _C_HID, 2 * _C_IN), lambda i: (0, 0)),
        pl.BlockSpec((2 * _C_HID, 1), lambda i: (0, 0)),
        pl.BlockSpec((_C_IN, 2 * _C_HID), lambda i: (0, 0)),
        pl.BlockSpec((_C_IN, 1), lambda i: (0, 0)),
    ]
    args = [sum3, eft, w1, b1, w2, b2]
    kwargs = {}
    if prev is not None:
        def body(sum3, eft, w1, b1, w2, b2, prev_ref, out_ref):
            _dense_body(sum3, eft, w1, b1, w2, b2, out_ref)
        in_specs.append(pl.BlockSpec(memory_space=pl.ANY))
        args.append(prev)
        kwargs["input_output_aliases"] = {6: 0}
    return pl.pallas_call(
        body,
        grid=(nblk,),
        in_specs=in_specs,
        out_specs=pl.BlockSpec((_C_IN, block_e), lambda i: (0, i + block0)),
        out_shape=jax.ShapeDtypeStruct((_C_IN, _E), jnp.float32),
        **kwargs,
    )(*args)


def kernel(node_features, edge_index, edge_features,
           W1a, b1a, W2a, b2a, W1b, b1b, W2b, b2b):
    pad = _E_PAD - _E
    src2d = jnp.concatenate(
        [edge_index[0], jnp.zeros((pad,), jnp.int32)]).reshape(_E_PAD // _GSLICE, _GSLICE)
    dst2d = jnp.concatenate(
        [edge_index[1], jnp.zeros((pad,), jnp.int32)]).reshape(_E_PAD // _GSLICE, _GSLICE)

    nsplit = 4
    e_span = _E_PAD // nsplit
    sums = [_sc_gather(node_features, src2d, dst2d, h, nsplit)
            .reshape(_C_IN, e_span // 128, 128) for h in range(nsplit)]

    w1 = jnp.zeros((2 * _C_HID, 2 * _C_IN), jnp.float32)
    w1 = w1.at[0:_C_HID, 0:_C_IN].set(W1a)
    w1 = w1.at[_C_HID:, _C_IN:].set(W1b)
    b1 = jnp.concatenate([b1a, b1b]).reshape(2 * _C_HID, 1)
    w2 = jnp.concatenate([W2a, W2b], axis=1)
    b2 = (b2a + b2b).reshape(_C_IN, 1)

    eft = edge_features.T
    block_e = 4096
    blocks_per_half = e_span // block_e
    total_blocks = pl.cdiv(_E, block_e)
    out = None
    for h in range(nsplit):
        block0 = h * blocks_per_half
        nblk = min(blocks_per_half, total_blocks - block0)
        out = _dense_half(sums[h], eft, w1, b1, w2, b2, out,
                          block0, nblk, block_e)
    return out.T


# speedup vs baseline: 9.4285x; 1.0036x over previous
import functools

import jax
import jax.numpy as jnp
from jax import lax
from jax.experimental import pallas as pl
from jax.experimental.pallas import tpu as pltpu
from jax.experimental.pallas import tpu_sc as plsc

_E = 800000
_C_IN = 16
_C_HID = 64
_NW = 32
_CHUNK = 640
_GSLICE = 128
_NR = _CHUNK // _GSLICE
_NSTEP = 40
_PER_W = _CHUNK * _NSTEP
_E_PAD = _PER_W * _NW


def _gather_body(nf_hbm, src_hbm, dst_hbm, out_hbm,
                 sidx0, sidx1, didx0, didx1, srows0, srows1, drows0, drows1,
                 tbuf0, tbuf1, shared_nf, si0, si1, sg0, sg1, sw0, sw1,
                 *, h, nstep, e_span):
    SIDX, DIDX = [sidx0, sidx1], [didx0, didx1]
    SROWS, DROWS = [srows0, srows1], [drows0, drows1]
    TBUF, SI, SG, SW = [tbuf0, tbuf1], [si0, si1], [sg0, sg1], [sw0, sw1]

    per_w = _CHUNK * nstep
    wid = lax.axis_index("s") * 2 + lax.axis_index("c")
    base = wid * per_w
    feat16 = jnp.arange(_C_IN, dtype=jnp.int32)
    half = nstep // 2

    def issue_idx(c, b):
        off = h * e_span + base + c * _CHUNK
        pltpu.async_copy(src_hbm.at[pl.ds(off, _CHUNK)], SIDX[b], SI[b])
        pltpu.async_copy(dst_hbm.at[pl.ds(off, _CHUNK)], DIDX[b], SI[b])

    def wait_idx(b):
        pltpu.make_async_copy(src_hbm.at[pl.ds(0, _CHUNK)], SIDX[b], SI[b]).wait()
        pltpu.make_async_copy(dst_hbm.at[pl.ds(0, _CHUNK)], DIDX[b], SI[b]).wait()

    def issue_gather(b):
        pltpu.async_copy(shared_nf.at[SIDX[b]], SROWS[b], SG[b])
        pltpu.async_copy(shared_nf.at[DIDX[b]], DROWS[b], SG[b])

    def wait_gather(b):
        pltpu.make_async_copy(nf_hbm.at[pl.ds(0, _CHUNK)], SROWS[b], SG[b]).wait()
        pltpu.make_async_copy(nf_hbm.at[pl.ds(0, _CHUNK)], DROWS[b], SG[b]).wait()

    def compute(b):
        @plsc.parallel_loop(0, _CHUNK, 1, unroll=8)
        def _(i):
            s = SROWS[b][i] + DROWS[b][i]
            plsc.store_scatter(TBUF[b], [feat16, jnp.full((_C_IN,), i, jnp.int32)], s)

    def issue_write(c, b):
        off = base + c * _CHUNK
        pltpu.async_copy(TBUF[b],
                         out_hbm.at[:, pl.ds(off, _CHUNK)], SW[b])

    def wait_write(b):
        pltpu.make_async_copy(TBUF[b],
                              out_hbm.at[:, pl.ds(0, _CHUNK)], SW[b]).wait()

    @pl.when(lax.axis_index("s") == 0)
    def _():
        pltpu.sync_copy(nf_hbm, shared_nf)
    plsc.subcore_barrier()

    issue_idx(0, 0)
    wait_idx(0)
    issue_gather(0)
    issue_idx(1, 1)

    def iter_g(g, carry):
        for b in (0, 1):
            c = 2 * g + b
            nb = 1 - b

            def stage_next():
                wait_idx(nb)
                issue_gather(nb)
            if b == 0:
                stage_next()
            else:
                pl.when(g < half - 1)(stage_next)

            wait_gather(b)
            pl.when(g >= 1)(lambda: wait_write(b))
            compute(b)
            issue_write(c, b)
            pl.when(g < half - 1)(lambda: issue_idx(c + 2, b))
        return carry

    lax.fori_loop(0, half, iter_g, 0)
    wait_write(0)
    wait_write(1)


def _sc_gather(node_features, src1d, dst1d, h, nsplit):
    e_span = _E_PAD // nsplit
    nstep = _NSTEP // nsplit
    mesh = plsc.VectorSubcoreMesh(core_axis_name="c", subcore_axis_name="s")
    f = pl.kernel(
        functools.partial(_gather_body, h=h, nstep=nstep, e_span=e_span),
        out_type=jax.ShapeDtypeStruct((_C_IN, e_span), jnp.float32),
        mesh=mesh,
        compiler_params=pltpu.CompilerParams(use_tc_tiling_on_sc=False,
                                             needs_layout_passes=False),
        scratch_types=(
            [pltpu.VMEM((_CHUNK,), jnp.int32) for _ in range(4)]
            + [pltpu.VMEM((_CHUNK, _C_IN), jnp.float32) for _ in range(4)]
            + [pltpu.VMEM((_C_IN, _CHUNK), jnp.float32) for _ in range(2)]
            + [pltpu.VMEM_SHARED((50000, _C_IN), jnp.float32)]
            + [pltpu.SemaphoreType.DMA for _ in range(6)]
        ),
    )
    return f(node_features, src1d, dst1d)


def _dense_body(sum3, eft, w1, b1, w2, b2, out_ref):
    ns = sum3[...].reshape(_C_IN, -1)
    x = jnp.concatenate([ns, eft[...]], axis=0)
    h = jnp.maximum(
        jnp.dot(w1[...], x, preferred_element_type=jnp.float32) + b1[...], 0.0)
    t = jnp.dot(w2[...], h, preferred_element_type=jnp.float32) + b2[...]
    mean = jnp.mean(t, axis=0, keepdims=True)
    var = jnp.mean((t - mean) ** 2, axis=0, keepdims=True)
    tn = (t - mean) * lax.rsqrt(var + 1e-5)
    out_ref[...] = eft[...] + jnp.maximum(tn, 0.0)


def _dense_half(sum3, eft, w1, b1, w2, b2, prev, block0, nblk, block_e=4096):
    body = _dense_body
    in_specs = [
        pl.BlockSpec((_C_IN, block_e // 128, 128), lambda i: (0, i, 0)),
        pl.BlockSpec((_C_IN, block_e), lambda i: (0, i + block0)),
        pl.BlockSpec((2 * _C_HID, 2 * _C_IN), lambda i: (0, 0)),
        pl.BlockSpec((2 * _C_HID, 1), lambda i: (0, 0)),
        pl.BlockSpec((_C_IN, 2 * _C_HID), lambda i: (0, 0)),
        pl.BlockSpec((_C_IN, 1), lambda i: (0, 0)),
    ]
    args = [sum3, eft, w1, b1, w2, b2]
    kwargs = {}
    if prev is not None:
        def body(sum3, eft, w1, b1, w2, b2, prev_ref, out_ref):
            _dense_body(sum3, eft, w1, b1, w2, b2, out_ref)
        in_specs.append(pl.BlockSpec(memory_space=pl.ANY))
        args.append(prev)
        kwargs["input_output_aliases"] = {6: 0}
    return pl.pallas_call(
        body,
        grid=(nblk,),
        in_specs=in_specs,
        out_specs=pl.BlockSpec((_C_IN, block_e), lambda i: (0, i + block0)),
        out_shape=jax.ShapeDtypeStruct((_C_IN, _E), jnp.float32),
        **kwargs,
    )(*args)


def kernel(node_features, edge_index, edge_features,
           W1a, b1a, W2a, b2a, W1b, b1b, W2b, b2b):
    pad = _E_PAD - _E
    src1d = jnp.concatenate([edge_index[0], jnp.zeros((pad,), jnp.int32)])
    dst1d = jnp.concatenate([edge_index[1], jnp.zeros((pad,), jnp.int32)])

    nsplit = 4
    e_span = _E_PAD // nsplit
    sums = [_sc_gather(node_features, src1d, dst1d, h, nsplit)
            .reshape(_C_IN, e_span // 128, 128) for h in range(nsplit)]

    w1 = jnp.zeros((2 * _C_HID, 2 * _C_IN), jnp.float32)
    w1 = w1.at[0:_C_HID, 0:_C_IN].set(W1a)
    w1 = w1.at[_C_HID:, _C_IN:].set(W1b)
    b1 = jnp.concatenate([b1a, b1b]).reshape(2 * _C_HID, 1)
    w2 = jnp.concatenate([W2a, W2b], axis=1)
    b2 = (b2a + b2b).reshape(_C_IN, 1)

    eft = edge_features.T
    block_e = 4096
    blocks_per_half = e_span // block_e
    total_blocks = pl.cdiv(_E, block_e)
    out = None
    for h in range(nsplit):
        block0 = h * blocks_per_half
        nblk = min(blocks_per_half, total_blocks - block0)
        out = _dense_half(sums[h], eft, w1, b1, w2, b2, out,
                          block0, nblk, block_e)
    return out.T


# speedup vs baseline: 9.4701x; 1.0044x over previous
import functools

import jax
import jax.numpy as jnp
from jax import lax
from jax.experimental import pallas as pl
from jax.experimental.pallas import tpu as pltpu
from jax.experimental.pallas import tpu_sc as plsc

_E = 800000
_C_IN = 16
_C_HID = 64
_NW = 32
_CHUNK = 800
_GSLICE = 128
_NR = _CHUNK // _GSLICE
_NSTEP = 32
_PER_W = _CHUNK * _NSTEP
_E_PAD = _PER_W * _NW


def _gather_body(nf_hbm, src_hbm, dst_hbm, out_hbm,
                 sidx0, sidx1, didx0, didx1, srows0, srows1, drows0, drows1,
                 tbuf0, tbuf1, shared_nf, si0, si1, sg0, sg1, sw0, sw1,
                 *, h, nstep, e_span):
    SIDX, DIDX = [sidx0, sidx1], [didx0, didx1]
    SROWS, DROWS = [srows0, srows1], [drows0, drows1]
    TBUF, SI, SG, SW = [tbuf0, tbuf1], [si0, si1], [sg0, sg1], [sw0, sw1]

    per_w = _CHUNK * nstep
    wid = lax.axis_index("s") * 2 + lax.axis_index("c")
    base = wid * per_w
    feat16 = jnp.arange(_C_IN, dtype=jnp.int32)
    half = nstep // 2

    def issue_idx(c, b):
        off = h * e_span + base + c * _CHUNK
        pltpu.async_copy(src_hbm.at[pl.ds(off, _CHUNK)], SIDX[b], SI[b])
        pltpu.async_copy(dst_hbm.at[pl.ds(off, _CHUNK)], DIDX[b], SI[b])

    def wait_idx(b):
        pltpu.make_async_copy(src_hbm.at[pl.ds(0, _CHUNK)], SIDX[b], SI[b]).wait()
        pltpu.make_async_copy(dst_hbm.at[pl.ds(0, _CHUNK)], DIDX[b], SI[b]).wait()

    def issue_gather(b):
        pltpu.async_copy(shared_nf.at[SIDX[b]], SROWS[b], SG[b])
        pltpu.async_copy(shared_nf.at[DIDX[b]], DROWS[b], SG[b])

    def wait_gather(b):
        pltpu.make_async_copy(nf_hbm.at[pl.ds(0, _CHUNK)], SROWS[b], SG[b]).wait()
        pltpu.make_async_copy(nf_hbm.at[pl.ds(0, _CHUNK)], DROWS[b], SG[b]).wait()

    def compute(b):
        @plsc.parallel_loop(0, _CHUNK, 1, unroll=8)
        def _(i):
            s = SROWS[b][i] + DROWS[b][i]
            plsc.store_scatter(TBUF[b], [feat16, jnp.full((_C_IN,), i, jnp.int32)], s)

    def issue_write(c, b):
        off = base + c * _CHUNK
        pltpu.async_copy(TBUF[b],
                         out_hbm.at[:, pl.ds(off, _CHUNK)], SW[b])

    def wait_write(b):
        pltpu.make_async_copy(TBUF[b],
                              out_hbm.at[:, pl.ds(0, _CHUNK)], SW[b]).wait()

    @pl.when(lax.axis_index("s") == 0)
    def _():
        pltpu.sync_copy(nf_hbm, shared_nf)
    plsc.subcore_barrier()

    issue_idx(0, 0)
    wait_idx(0)
    issue_gather(0)
    issue_idx(1, 1)

    def iter_g(g, carry):
        for b in (0, 1):
            c = 2 * g + b
            nb = 1 - b

            def stage_next():
                wait_idx(nb)
                issue_gather(nb)
            if b == 0:
                stage_next()
            else:
                pl.when(g < half - 1)(stage_next)

            wait_gather(b)
            pl.when(g >= 1)(lambda: wait_write(b))
            compute(b)
            issue_write(c, b)
            pl.when(g < half - 1)(lambda: issue_idx(c + 2, b))
        return carry

    lax.fori_loop(0, half, iter_g, 0)
    wait_write(0)
    wait_write(1)


def _sc_gather(node_features, src1d, dst1d, h, nsplit):
    e_span = _E_PAD // nsplit
    nstep = _NSTEP // nsplit
    mesh = plsc.VectorSubcoreMesh(core_axis_name="c", subcore_axis_name="s")
    f = pl.kernel(
        functools.partial(_gather_body, h=h, nstep=nstep, e_span=e_span),
        out_type=jax.ShapeDtypeStruct((_C_IN, e_span), jnp.float32),
        mesh=mesh,
        compiler_params=pltpu.CompilerParams(use_tc_tiling_on_sc=False,
                                             needs_layout_passes=False),
        scratch_types=(
            [pltpu.VMEM((_CHUNK,), jnp.int32) for _ in range(4)]
            + [pltpu.VMEM((_CHUNK, _C_IN), jnp.float32) for _ in range(4)]
            + [pltpu.VMEM((_C_IN, _CHUNK), jnp.float32) for _ in range(2)]
            + [pltpu.VMEM_SHARED((50000, _C_IN), jnp.float32)]
            + [pltpu.SemaphoreType.DMA for _ in range(6)]
        ),
    )
    return f(node_features, src1d, dst1d)


def _dense_body(sum3, eft, w1, b1, w2, b2, out_ref):
    ns = sum3[...].reshape(_C_IN, -1)
    x = jnp.concatenate([ns, eft[...]], axis=0)
    h = jnp.maximum(
        jnp.dot(w1[...], x, preferred_element_type=jnp.float32) + b1[...], 0.0)
    t = jnp.dot(w2[...], h, preferred_element_type=jnp.float32) + b2[...]
    mean = jnp.mean(t, axis=0, keepdims=True)
    var = jnp.mean((t - mean) ** 2, axis=0, keepdims=True)
    tn = (t - mean) * lax.rsqrt(var + 1e-5)
    out_ref[...] = eft[...] + jnp.maximum(tn, 0.0)


def _dense_half(sum3, eft, w1, b1, w2, b2, prev, block0, nblk, block_e=4096):
    body = _dense_body
    in_specs = [
        pl.BlockSpec((_C_IN, block_e // 128, 128), lambda i: (0, i, 0)),
        pl.BlockSpec((_C_IN, block_e), lambda i: (0, i + block0)),
        pl.BlockSpec((2 * _C_HID, 2 * _C_IN), lambda i: (0, 0)),
        pl.BlockSpec((2 * _C_HID, 1), lambda i: (0, 0)),
        pl.BlockSpec((_C_IN, 2 * _C_HID), lambda i: (0, 0)),
        pl.BlockSpec((_C_IN, 1), lambda i: (0, 0)),
    ]
    args = [sum3, eft, w1, b1, w2, b2]
    kwargs = {}
    if prev is not None:
        def body(sum3, eft, w1, b1, w2, b2, prev_ref, out_ref):
            _dense_body(sum3, eft, w1, b1, w2, b2, out_ref)
        in_specs.append(pl.BlockSpec(memory_space=pl.ANY))
        args.append(prev)
        kwargs["input_output_aliases"] = {6: 0}
    return pl.pallas_call(
        body,
        grid=(nblk,),
        in_specs=in_specs,
        out_specs=pl.BlockSpec((_C_IN, block_e), lambda i: (0, i + block0)),
        out_shape=jax.ShapeDtypeStruct((_C_IN, _E), jnp.float32),
        **kwargs,
    )(*args)


def kernel(node_features, edge_index, edge_features,
           W1a, b1a, W2a, b2a, W1b, b1b, W2b, b2b):
    pad = _E_PAD - _E
    src1d = jnp.concatenate([edge_index[0], jnp.zeros((pad,), jnp.int32)])
    dst1d = jnp.concatenate([edge_index[1], jnp.zeros((pad,), jnp.int32)])

    nsplit = 4
    e_span = _E_PAD // nsplit
    sums = [_sc_gather(node_features, src1d, dst1d, h, nsplit)
            .reshape(_C_IN, e_span // 128, 128) for h in range(nsplit)]

    w1 = jnp.zeros((2 * _C_HID, 2 * _C_IN), jnp.float32)
    w1 = w1.at[0:_C_HID, 0:_C_IN].set(W1a)
    w1 = w1.at[_C_HID:, _C_IN:].set(W1b)
    b1 = jnp.concatenate([b1a, b1b]).reshape(2 * _C_HID, 1)
    w2 = jnp.concatenate([W2a, W2b], axis=1)
    b2 = (b2a + b2b).reshape(_C_IN, 1)

    eft = edge_features.T
    block_e = 4096
    blocks_per_half = e_span // block_e
    total_blocks = pl.cdiv(_E, block_e)
    out = None
    for h in range(nsplit):
        block0 = h * blocks_per_half
        nblk = min(blocks_per_half, total_blocks - block0)
        out = _dense_half(sums[h], eft, w1, b1, w2, b2, out,
                          block0, nblk, block_e)
    return out.T
